# baseline (speedup 1.0000x reference)
import jax
import jax.numpy as jnp
from jax.experimental import pallas as pl
from jax.experimental.pallas import tpu as pltpu

TOTAL = 16384
D = 2048
SEG = 2048
NSEG = TOTAL // SEG
BN = 512
N_TILES = D // BN


def _fused_body(x_ref, w_ref, b_ref, o_ref, xc_ref):
    @pl.when(pl.program_id(1) == 0)
    def _():
        xm = jnp.mean(x_ref[...], axis=0, keepdims=True)
        xc_ref[...] = (x_ref[...] - xm).astype(jnp.bfloat16)

    o_ref[...] = jax.lax.dot_general(
        xc_ref[...], w_ref[...],
        dimension_numbers=(((1,), (1,)), ((), ())),
        preferred_element_type=jnp.float32,
    ) + b_ref[...]


def kernel(x, W, b, l):
    b_eff = (b + (jnp.asarray(l) - SEG).astype(jnp.float32)).reshape(1, D)
    W_bf = W.astype(jnp.bfloat16)

    out = pl.pallas_call(
        _fused_body,
        grid=(NSEG, N_TILES),
        in_specs=[
            pl.BlockSpec((SEG, D), lambda s, n: (s, 0)),
            pl.BlockSpec((BN, D), lambda s, n: (n, 0)),
            pl.BlockSpec((1, BN), lambda s, n: (0, n)),
        ],
        out_specs=pl.BlockSpec((SEG, BN), lambda s, n: (s, n)),
        out_shape=jax.ShapeDtypeStruct((TOTAL, D), jnp.float32),
        scratch_shapes=[pltpu.VMEM((SEG, D), jnp.bfloat16)],
    )(x, W_bf, b_eff)
    return out


# speedup vs baseline: 1.2523x; 1.2523x over previous
import jax
import jax.numpy as jnp
from jax.experimental import pallas as pl
from jax.experimental.pallas import tpu as pltpu

TOTAL = 16384
D = 2048
SEG = 2048
NSEG = TOTAL // SEG
BM = 1024
M_TILES = SEG // BM


def _fused_body(x_ref, w_ref, b_ref, o_ref, xm_ref):
    m = pl.program_id(1)

    @pl.when(m == 0)
    def _():
        xm_ref[...] = jnp.mean(x_ref[...], axis=0, keepdims=True)

    xc = (x_ref[pl.ds(m * BM, BM), :] - xm_ref[...]).astype(jnp.bfloat16)
    o_ref[...] = jax.lax.dot_general(
        xc, w_ref[...],
        dimension_numbers=(((1,), (1,)), ((), ())),
        preferred_element_type=jnp.float32,
    ) + b_ref[...]


def kernel(x, W, b, l):
    b_eff = (b + (jnp.asarray(l) - SEG).astype(jnp.float32)).reshape(1, D)
    W_bf = W.astype(jnp.bfloat16)

    out = pl.pallas_call(
        _fused_body,
        grid=(NSEG, M_TILES),
        in_specs=[
            pl.BlockSpec((SEG, D), lambda s, m: (s, 0)),
            pl.BlockSpec((D, D), lambda s, m: (0, 0)),
            pl.BlockSpec((1, D), lambda s, m: (0, 0)),
        ],
        out_specs=pl.BlockSpec((BM, D), lambda s, m: (s * M_TILES + m, 0)),
        out_shape=jax.ShapeDtypeStruct((TOTAL, D), jnp.float32),
        scratch_shapes=[pltpu.VMEM((1, D), jnp.float32)],
        compiler_params=pltpu.CompilerParams(
            vmem_limit_bytes=64 * 1024 * 1024,
        ),
    )(x, W_bf, b_eff)
    return out
